# baseline (speedup 1.0000x reference)
import functools

import jax
import jax.numpy as jnp
from jax import lax
from jax.experimental import pallas as pl
from jax.experimental.pallas import tpu as pltpu
from jax.experimental.pallas import tpu_sc as plsc

N = 10000
E = 320000
D = 128
D_MLP = 64

NC = 2
NS = 16
NW = NC * NS

B = 128
EPW = E // NW
C = 80
EPW_P = C * B
HALF = C // 2

ROWS_PER_TILE = 632
NPAD = NS * ROWS_PER_TILE

NBUF = 2


def _sc_aggregate_build():
  mesh = plsc.VectorSubcoreMesh(core_axis_name="c", subcore_axis_name="s")

  @functools.partial(
      pl.kernel,
      out_type=jax.ShapeDtypeStruct((NC, NPAD, D), jnp.float32),
      mesh=mesh,
      scratch_types=[
          pltpu.VMEM((HALF, B), jnp.int32),
          pltpu.VMEM((HALF, B), jnp.int32),
          pltpu.VMEM((NBUF, B, D), jnp.float32),
          pltpu.VMEM_SHARED((NPAD, D), jnp.float32),
          pltpu.SemaphoreType.DMA,
          pltpu.SemaphoreType.DMA,
      ],
  )
  def agg_kernel(table_hbm, edges_hbm, out_hbm,
                 src_v, dst_v, rows_v, acc_sh, sem0, sem1):
    cid = lax.axis_index("c")
    sid = lax.axis_index("s")
    wid = cid * NS + sid
    sems = (sem0, sem1)

    idx0 = pltpu.async_copy(edges_hbm.at[0, wid, 0], src_v, sem0)
    idx1 = pltpu.async_copy(edges_hbm.at[1, wid, 0], dst_v, sem1)

    zeros16 = jnp.zeros((16,), jnp.float32)

    @pl.loop(0, B)
    def _(i):
      for j in range(D // 16):
        rows_v[0, i, pl.ds(j * 16, 16)] = zeros16

    row0 = sid * ROWS_PER_TILE
    off = 0
    remaining = ROWS_PER_TILE
    while remaining > 0:
      sz = min(B, remaining)
      pltpu.sync_copy(rows_v.at[0, pl.ds(0, sz)],
                      acc_sh.at[pl.ds(row0 + off, sz)])
      off += sz
      remaining -= sz
    idx0.wait()
    idx1.wait()
    plsc.subcore_barrier()

    def start_gather(j, b):
      pltpu.async_copy(table_hbm.at[src_v.at[j]], rows_v.at[b], sems[b])

    def wait_gather(b):
      pltpu.make_async_copy(table_hbm.at[src_v.at[0]], rows_v.at[b],
                            sems[b]).wait()

    for h in range(2):
      if h > 0:
        pltpu.sync_copy(edges_hbm.at[0, wid, h], src_v)
        pltpu.sync_copy(edges_hbm.at[1, wid, h], dst_v)

      for b in range(NBUF):
        start_gather(b, b)

      @pl.loop(0, HALF // NBUF)
      def _(jj):
        j0 = jj * NBUF
        for b in range(NBUF):
          j = j0 + b
          wait_gather(b)
          pltpu.sync_copy(rows_v.at[b], acc_sh.at[dst_v.at[j]], add=True)

          @pl.when(j + NBUF < HALF)
          def _():
            start_gather(j + NBUF, b)

    plsc.subcore_barrier()
    pltpu.sync_copy(acc_sh.at[pl.ds(row0, ROWS_PER_TILE)],
                    out_hbm.at[cid, pl.ds(row0, ROWS_PER_TILE)])

  return agg_kernel


_sc_aggregate = _sc_aggregate_build()


_R = 1000


def _mmt(a, w):
  return lax.dot_general(a, w, (((1,), (1,)), ((), ())),
                         preferred_element_type=jnp.float32)


def _root_body(x_ref, w_ref, b_ref, o_ref):
  o_ref[...] = _mmt(x_ref[...], w_ref[...]) + b_ref[...]


def _root(x, w, b):
  return pl.pallas_call(
      _root_body,
      out_shape=jax.ShapeDtypeStruct((N, D), jnp.float32),
      grid=(N // _R,),
      in_specs=[
          pl.BlockSpec((_R, D), lambda i: (i, 0)),
          pl.BlockSpec((D, D), lambda i: (0, 0)),
          pl.BlockSpec((1, D), lambda i: (0, 0)),
      ],
      out_specs=pl.BlockSpec((_R, D), lambda i: (i, 0)),
  )(x, w, b)


def _combine1_body(agg_ref, t_ref, wr_ref, o_ref):
  a = agg_ref[0] + agg_ref[1]
  o_ref[...] = jnp.maximum(_mmt(a, wr_ref[...]) + t_ref[...], 0.0)


def _combine1(agg, t, wr):
  return pl.pallas_call(
      _combine1_body,
      out_shape=jax.ShapeDtypeStruct((N, D), jnp.float32),
      grid=(N // _R,),
      in_specs=[
          pl.BlockSpec((NC, _R, D), lambda i: (0, i, 0)),
          pl.BlockSpec((_R, D), lambda i: (i, 0)),
          pl.BlockSpec((D, D), lambda i: (0, 0)),
      ],
      out_specs=pl.BlockSpec((_R, D), lambda i: (i, 0)),
  )(agg, t, wr)


def _combine2_body(agg_ref, t_ref, w2r_ref,
                   wm1_ref, bm1_ref, wm2_ref, bm2_ref, o_ref):
  a = agg_ref[0] + agg_ref[1]
  h2 = jnp.maximum(_mmt(a, w2r_ref[...]) + t_ref[...], 0.0)
  m = jnp.maximum(
      jnp.dot(h2, wm1_ref[...], preferred_element_type=jnp.float32)
      + bm1_ref[...], 0.0)
  o_ref[...] = (jnp.dot(m, wm2_ref[...], preferred_element_type=jnp.float32)
                + bm2_ref[...])


def _combine2(agg, t, w2r, wm1, bm1, wm2, bm2):
  return pl.pallas_call(
      _combine2_body,
      out_shape=jax.ShapeDtypeStruct((N, 1), jnp.float32),
      grid=(N // _R,),
      in_specs=[
          pl.BlockSpec((NC, _R, D), lambda i: (0, i, 0)),
          pl.BlockSpec((_R, D), lambda i: (i, 0)),
          pl.BlockSpec((D, D), lambda i: (0, 0)),
          pl.BlockSpec((D, D_MLP), lambda i: (0, 0)),
          pl.BlockSpec((1, D_MLP), lambda i: (0, 0)),
          pl.BlockSpec((D_MLP, 1), lambda i: (0, 0)),
          pl.BlockSpec((1, 1), lambda i: (0, 0)),
      ],
      out_specs=pl.BlockSpec((_R, 1), lambda i: (i, 0)),
  )(agg, t, w2r, wm1, bm1, wm2, bm2)


def kernel(x, edge_index, W1r, b1, W1o, W2r, b2, W2o, Wm1, bm1, Wm2, bm2):
  edges = edge_index.astype(jnp.int32).reshape(2, NW, EPW)
  pad = EPW_P - EPW
  pad_ids = jnp.arange(NW * pad, dtype=jnp.int32).reshape(1, NW, pad)
  pads = jnp.concatenate([pad_ids % N, N + pad_ids % (NPAD - N)], axis=0)
  edges_p = jnp.concatenate([edges, pads], axis=2).reshape(2, NW, 2, HALF, B)

  agg1 = _sc_aggregate(x, edges_p)
  t1 = _root(x, W1o, b1.reshape(1, D))
  h1 = _combine1(agg1, t1, W1r)
  agg2 = _sc_aggregate(h1, edges_p)
  t2 = _root(h1, W2o, b2.reshape(1, D))
  out = _combine2(agg2, t2, W2r, Wm1.T, bm1.reshape(1, D_MLP), Wm2.T,
                  bm2.reshape(1, 1))
  return out


# speedup vs baseline: 1.0218x; 1.0218x over previous
import functools

import jax
import jax.numpy as jnp
from jax import lax
from jax.experimental import pallas as pl
from jax.experimental.pallas import tpu as pltpu
from jax.experimental.pallas import tpu_sc as plsc

N = 10000
E = 320000
D = 128
D_MLP = 64

NC = 2
NS = 16
NW = NC * NS

B = 128
EPW = E // NW
C = 80
EPW_P = C * B
HALF = C // 2

ROWS_PER_TILE = 632
NPAD = NS * ROWS_PER_TILE

NBUF = 2


def _sc_aggregate_build():
  mesh = plsc.VectorSubcoreMesh(core_axis_name="c", subcore_axis_name="s")

  @functools.partial(
      pl.kernel,
      out_type=jax.ShapeDtypeStruct((NC, NPAD, D), jnp.float32),
      mesh=mesh,
      scratch_types=[
          pltpu.VMEM((HALF, B), jnp.int32),
          pltpu.VMEM((HALF, B), jnp.int32),
          pltpu.VMEM((NBUF, B, D), jnp.float32),
          pltpu.VMEM_SHARED((NPAD, D), jnp.float32),
          pltpu.SemaphoreType.DMA,
          pltpu.SemaphoreType.DMA,
      ],
  )
  def agg_kernel(table_hbm, edges_hbm, out_hbm,
                 src_v, dst_v, rows_v, acc_sh, sem0, sem1):
    cid = lax.axis_index("c")
    sid = lax.axis_index("s")
    wid = cid * NS + sid
    sems = (sem0, sem1)

    idx0 = pltpu.async_copy(edges_hbm.at[0, wid, 0], src_v, sem0)
    idx1 = pltpu.async_copy(edges_hbm.at[1, wid, 0], dst_v, sem1)

    zeros16 = jnp.zeros((16,), jnp.float32)

    @pl.loop(0, B)
    def _(i):
      for j in range(D // 16):
        rows_v[0, i, pl.ds(j * 16, 16)] = zeros16

    row0 = sid * ROWS_PER_TILE
    off = 0
    remaining = ROWS_PER_TILE
    while remaining > 0:
      sz = min(B, remaining)
      pltpu.sync_copy(rows_v.at[0, pl.ds(0, sz)],
                      acc_sh.at[pl.ds(row0 + off, sz)])
      off += sz
      remaining -= sz
    idx0.wait()
    idx1.wait()
    plsc.subcore_barrier()

    def start_gather(j, b):
      pltpu.async_copy(table_hbm.at[src_v.at[j]], rows_v.at[b], sems[b])

    def wait_gather(b):
      pltpu.make_async_copy(table_hbm.at[src_v.at[0]], rows_v.at[b],
                            sems[b]).wait()

    for h in range(2):
      if h > 0:
        pltpu.sync_copy(edges_hbm.at[0, wid, h], src_v)
        pltpu.sync_copy(edges_hbm.at[1, wid, h], dst_v)

      for b in range(NBUF):
        start_gather(b, b)

      @pl.loop(0, HALF // NBUF)
      def _(jj):
        j0 = jj * NBUF
        for b in range(NBUF):
          j = j0 + b
          wait_gather(b)
          pltpu.sync_copy(rows_v.at[b], acc_sh.at[dst_v.at[j]], add=True)

          @pl.when(j + NBUF < HALF)
          def _():
            start_gather(j + NBUF, b)

    plsc.subcore_barrier()
    pltpu.sync_copy(acc_sh.at[pl.ds(row0, ROWS_PER_TILE)],
                    out_hbm.at[cid, pl.ds(row0, ROWS_PER_TILE)])

  return agg_kernel


_sc_aggregate = _sc_aggregate_build()


_R = 2000


def _mmt(a, w):
  return lax.dot_general(a, w, (((1,), (1,)), ((), ())),
                         preferred_element_type=jnp.float32)


def _root_body(x_ref, w_ref, b_ref, o_ref):
  o_ref[...] = _mmt(x_ref[...], w_ref[...]) + b_ref[...]


def _root(x, w, b):
  return pl.pallas_call(
      _root_body,
      out_shape=jax.ShapeDtypeStruct((N, D), jnp.float32),
      grid=(N // _R,),
      in_specs=[
          pl.BlockSpec((_R, D), lambda i: (i, 0)),
          pl.BlockSpec((D, D), lambda i: (0, 0)),
          pl.BlockSpec((1, D), lambda i: (0, 0)),
      ],
      out_specs=pl.BlockSpec((_R, D), lambda i: (i, 0)),
  )(x, w, b)


def _combine1_body(agg_ref, t_ref, wr_ref, o_ref):
  a = agg_ref[0] + agg_ref[1]
  o_ref[...] = jnp.maximum(_mmt(a, wr_ref[...]) + t_ref[...], 0.0)


def _combine1(agg, t, wr):
  return pl.pallas_call(
      _combine1_body,
      out_shape=jax.ShapeDtypeStruct((N, D), jnp.float32),
      grid=(N // _R,),
      in_specs=[
          pl.BlockSpec((NC, _R, D), lambda i: (0, i, 0)),
          pl.BlockSpec((_R, D), lambda i: (i, 0)),
          pl.BlockSpec((D, D), lambda i: (0, 0)),
      ],
      out_specs=pl.BlockSpec((_R, D), lambda i: (i, 0)),
  )(agg, t, wr)


def _combine2_body(agg_ref, t_ref, w2r_ref,
                   wm1_ref, bm1_ref, wm2_ref, bm2_ref, o_ref):
  a = agg_ref[0] + agg_ref[1]
  h2 = jnp.maximum(_mmt(a, w2r_ref[...]) + t_ref[...], 0.0)
  m = jnp.maximum(
      jnp.dot(h2, wm1_ref[...], preferred_element_type=jnp.float32)
      + bm1_ref[...], 0.0)
  o_ref[...] = (jnp.dot(m, wm2_ref[...], preferred_element_type=jnp.float32)
                + bm2_ref[...])


def _combine2(agg, t, w2r, wm1, bm1, wm2, bm2):
  return pl.pallas_call(
      _combine2_body,
      out_shape=jax.ShapeDtypeStruct((N, 1), jnp.float32),
      grid=(N // _R,),
      in_specs=[
          pl.BlockSpec((NC, _R, D), lambda i: (0, i, 0)),
          pl.BlockSpec((_R, D), lambda i: (i, 0)),
          pl.BlockSpec((D, D), lambda i: (0, 0)),
          pl.BlockSpec((D, D_MLP), lambda i: (0, 0)),
          pl.BlockSpec((1, D_MLP), lambda i: (0, 0)),
          pl.BlockSpec((D_MLP, 1), lambda i: (0, 0)),
          pl.BlockSpec((1, 1), lambda i: (0, 0)),
      ],
      out_specs=pl.BlockSpec((_R, 1), lambda i: (i, 0)),
  )(agg, t, w2r, wm1, bm1, wm2, bm2)


def kernel(x, edge_index, W1r, b1, W1o, W2r, b2, W2o, Wm1, bm1, Wm2, bm2):
  edges = edge_index.astype(jnp.int32).reshape(2, NW, EPW)
  pad = EPW_P - EPW
  pad_ids = jnp.arange(NW * pad, dtype=jnp.int32).reshape(1, NW, pad)
  pads = jnp.concatenate([pad_ids % N, N + pad_ids % (NPAD - N)], axis=0)
  edges_p = jnp.concatenate([edges, pads], axis=2).reshape(2, NW, 2, HALF, B)

  agg1 = _sc_aggregate(x, edges_p)
  t1 = _root(x, W1o, b1.reshape(1, D))
  h1 = _combine1(agg1, t1, W1r)
  agg2 = _sc_aggregate(h1, edges_p)
  t2 = _root(h1, W2o, b2.reshape(1, D))
  out = _combine2(agg2, t2, W2r, Wm1.T, bm1.reshape(1, D_MLP), Wm2.T,
                  bm2.reshape(1, 1))
  return out


# speedup vs baseline: 1.0218x; 1.0000x over previous
import functools

import jax
import jax.numpy as jnp
from jax import lax
from jax.experimental import pallas as pl
from jax.experimental.pallas import tpu as pltpu
from jax.experimental.pallas import tpu_sc as plsc

N = 10000
E = 320000
D = 128
D_MLP = 64

NC = 2
NS = 16
NW = NC * NS

B = 128
EPW = E // NW
C = 80
EPW_P = C * B
HALF = C // 2

ROWS_PER_TILE = 632
NPAD = NS * ROWS_PER_TILE

NBUF = 2


def _sc_aggregate_build():
  mesh = plsc.VectorSubcoreMesh(core_axis_name="c", subcore_axis_name="s")

  @functools.partial(
      pl.kernel,
      out_type=jax.ShapeDtypeStruct((NC, NPAD, D), jnp.float32),
      mesh=mesh,
      scratch_types=[
          pltpu.VMEM((HALF, B), jnp.int32),
          pltpu.VMEM((HALF, B), jnp.int32),
          pltpu.VMEM((NBUF, B, D), jnp.float32),
          pltpu.VMEM_SHARED((NPAD, D), jnp.float32),
          pltpu.SemaphoreType.DMA,
          pltpu.SemaphoreType.DMA,
      ],
  )
  def agg_kernel(table_hbm, edges_hbm, out_hbm,
                 src_v, dst_v, rows_v, acc_sh, sem0, sem1):
    cid = lax.axis_index("c")
    sid = lax.axis_index("s")
    wid = cid * NS + sid
    sems = (sem0, sem1)

    idx0 = pltpu.async_copy(edges_hbm.at[0, wid, 0], src_v, sem0)
    idx1 = pltpu.async_copy(edges_hbm.at[1, wid, 0], dst_v, sem1)

    zeros16 = jnp.zeros((16,), jnp.float32)

    @pl.loop(0, B)
    def _(i):
      for j in range(D // 16):
        rows_v[0, i, pl.ds(j * 16, 16)] = zeros16

    idx0.wait()
    idx1.wait()

    row0 = sid * ROWS_PER_TILE
    zcopies = []
    off = 0
    remaining = ROWS_PER_TILE
    while remaining > 0:
      sz = min(B, remaining)
      zcopies.append(
          pltpu.async_copy(rows_v.at[0, pl.ds(0, sz)],
                           acc_sh.at[pl.ds(row0 + off, sz)],
                           sems[len(zcopies) % 2]))
      off += sz
      remaining -= sz
    for zc in zcopies:
      zc.wait()
    plsc.subcore_barrier()

    def start_gather(j, b):
      pltpu.async_copy(table_hbm.at[src_v.at[j]], rows_v.at[b], sems[b])

    def wait_gather(b):
      pltpu.make_async_copy(table_hbm.at[src_v.at[0]], rows_v.at[b],
                            sems[b]).wait()

    for h in range(2):
      if h > 0:
        pltpu.sync_copy(edges_hbm.at[0, wid, h], src_v)
        pltpu.sync_copy(edges_hbm.at[1, wid, h], dst_v)

      for b in range(NBUF):
        start_gather(b, b)

      @pl.loop(0, HALF // NBUF)
      def _(jj):
        j0 = jj * NBUF
        for b in range(NBUF):
          j = j0 + b
          wait_gather(b)
          pltpu.sync_copy(rows_v.at[b], acc_sh.at[dst_v.at[j]], add=True)

          @pl.when(j + NBUF < HALF)
          def _():
            start_gather(j + NBUF, b)

    plsc.subcore_barrier()
    pltpu.sync_copy(acc_sh.at[pl.ds(row0, ROWS_PER_TILE)],
                    out_hbm.at[cid, pl.ds(row0, ROWS_PER_TILE)])

  return agg_kernel


_sc_aggregate = _sc_aggregate_build()


_R = 2000


def _mmt(a, w):
  return lax.dot_general(a, w, (((1,), (1,)), ((), ())),
                         preferred_element_type=jnp.float32)


def _root_body(x_ref, w_ref, b_ref, o_ref):
  o_ref[...] = _mmt(x_ref[...], w_ref[...]) + b_ref[...]


def _root(x, w, b):
  return pl.pallas_call(
      _root_body,
      out_shape=jax.ShapeDtypeStruct((N, D), jnp.float32),
      grid=(N // _R,),
      in_specs=[
          pl.BlockSpec((_R, D), lambda i: (i, 0)),
          pl.BlockSpec((D, D), lambda i: (0, 0)),
          pl.BlockSpec((1, D), lambda i: (0, 0)),
      ],
      out_specs=pl.BlockSpec((_R, D), lambda i: (i, 0)),
  )(x, w, b)


def _combine1_body(agg_ref, t_ref, wr_ref, o_ref):
  a = agg_ref[0] + agg_ref[1]
  o_ref[...] = jnp.maximum(_mmt(a, wr_ref[...]) + t_ref[...], 0.0)


def _combine1(agg, t, wr):
  return pl.pallas_call(
      _combine1_body,
      out_shape=jax.ShapeDtypeStruct((N, D), jnp.float32),
      grid=(N // _R,),
      in_specs=[
          pl.BlockSpec((NC, _R, D), lambda i: (0, i, 0)),
          pl.BlockSpec((_R, D), lambda i: (i, 0)),
          pl.BlockSpec((D, D), lambda i: (0, 0)),
      ],
      out_specs=pl.BlockSpec((_R, D), lambda i: (i, 0)),
  )(agg, t, wr)


def _combine2_body(agg_ref, t_ref, w2r_ref,
                   wm1_ref, bm1_ref, wm2_ref, bm2_ref, o_ref):
  a = agg_ref[0] + agg_ref[1]
  h2 = jnp.maximum(_mmt(a, w2r_ref[...]) + t_ref[...], 0.0)
  m = jnp.maximum(
      jnp.dot(h2, wm1_ref[...], preferred_element_type=jnp.float32)
      + bm1_ref[...], 0.0)
  o_ref[...] = (jnp.dot(m, wm2_ref[...], preferred_element_type=jnp.float32)
                + bm2_ref[...])


def _combine2(agg, t, w2r, wm1, bm1, wm2, bm2):
  return pl.pallas_call(
      _combine2_body,
      out_shape=jax.ShapeDtypeStruct((N, 1), jnp.float32),
      grid=(N // _R,),
      in_specs=[
          pl.BlockSpec((NC, _R, D), lambda i: (0, i, 0)),
          pl.BlockSpec((_R, D), lambda i: (i, 0)),
          pl.BlockSpec((D, D), lambda i: (0, 0)),
          pl.BlockSpec((D, D_MLP), lambda i: (0, 0)),
          pl.BlockSpec((1, D_MLP), lambda i: (0, 0)),
          pl.BlockSpec((D_MLP, 1), lambda i: (0, 0)),
          pl.BlockSpec((1, 1), lambda i: (0, 0)),
      ],
      out_specs=pl.BlockSpec((_R, 1), lambda i: (i, 0)),
  )(agg, t, w2r, wm1, bm1, wm2, bm2)


def kernel(x, edge_index, W1r, b1, W1o, W2r, b2, W2o, Wm1, bm1, Wm2, bm2):
  edges = edge_index.astype(jnp.int32).reshape(2, NW, EPW)
  pad = EPW_P - EPW
  pad_ids = jnp.arange(NW * pad, dtype=jnp.int32).reshape(1, NW, pad)
  pads = jnp.concatenate([pad_ids % N, N + pad_ids % (NPAD - N)], axis=0)
  edges_p = jnp.concatenate([edges, pads], axis=2).reshape(2, NW, 2, HALF, B)

  agg1 = _sc_aggregate(x, edges_p)
  t1 = _root(x, W1o, b1.reshape(1, D))
  h1 = _combine1(agg1, t1, W1r)
  agg2 = _sc_aggregate(h1, edges_p)
  t2 = _root(h1, W2o, b2.reshape(1, D))
  out = _combine2(agg2, t2, W2r, Wm1.T, bm1.reshape(1, D_MLP), Wm2.T,
                  bm2.reshape(1, 1))
  return out
